# baseline (speedup 1.0000x reference)
import functools
import math

import numpy as np
import jax
import jax.numpy as jnp
from jax.experimental import pallas as pl

HI = jax.lax.Precision.HIGHEST

_EXPERT_TYPES = ['conv', 'fourier', 'mlp', 'conv', 'fourier', 'mlp', 'conv', 'fourier']
_SPARSE_EIDS = [0, 2, 3, 5, 6]
_FOURIER_EIDS = [1, 4, 7]
_JMAP = [0, 5, 1, 2, 6, 3, 4, 7]


def _gelu(h):
    return h * 0.5 * (1.0 + jax.lax.erf(h / np.float32(np.sqrt(2.0))))


def _router_body(xf, wp, bp, w01_ref, jp_ref, aux_ref, *, E, topk):
    x = xf[...]
    logits = jax.lax.dot_general(x, wp[...], (((1,), (0,)), ((), ())),
                                 preferred_element_type=jnp.float32) + bp[...]
    lanes = jax.lax.broadcasted_iota(jnp.int32, logits.shape, 1)
    neg = jnp.float32(-1e30)
    logits = jnp.where(lanes < E, logits, neg)
    m = jnp.max(logits, axis=-1, keepdims=True)
    ex = jnp.where(lanes < E, jnp.exp(logits - m), 0.0)
    probs = ex / jnp.sum(ex, axis=-1, keepdims=True)
    m1 = jnp.max(probs, axis=-1, keepdims=True)
    a1 = jnp.min(jnp.where(probs >= m1, lanes, E), axis=-1, keepdims=True)
    p2 = jnp.where(lanes == a1, neg, probs)
    m2 = jnp.max(p2, axis=-1, keepdims=True)
    a2 = jnp.min(jnp.where(p2 >= m2, lanes, E), axis=-1, keepdims=True)
    denom = m1 + m2
    w0 = m1 / denom
    w1 = m2 / denom
    col = lanes
    w01_ref[...] = jnp.where(col == 0, w0, 0.0) + jnp.where(col == 1, w1, 0.0)
    j1 = jnp.zeros_like(a1)
    j2 = jnp.zeros_like(a2)
    for e in range(E):
        j1 = j1 + jnp.where(a1 == e, _JMAP[e], 0)
        j2 = j2 + jnp.where(a2 == e, _JMAP[e], 0)
    jp_ref[...] = (jnp.where(col == 0, j1, 0) + jnp.where(col == 1, j2, 0)
                   ).astype(jnp.int32)
    ntok = x.shape[0]
    me = jnp.sum(probs, axis=0) / ntok
    cnt = jnp.sum((lanes == a1).astype(jnp.float32)
                  + (lanes == a2).astype(jnp.float32), axis=0)
    ce = cnt / (ntok * topk)
    aux_ref[...] = jnp.reshape(E * jnp.sum(me * ce), (1, 1))


def _router_call(xf, router_w, router_b, E, topk):
    M, D = xf.shape
    wp = jnp.zeros((D, 128), jnp.float32).at[:, :E].set(router_w)
    bp = jnp.zeros((1, 128), jnp.float32).at[0, :E].set(router_b)
    out = pl.pallas_call(
        functools.partial(_router_body, E=E, topk=topk),
        out_shape=(jax.ShapeDtypeStruct((M, 128), jnp.float32),
                   jax.ShapeDtypeStruct((M, 128), jnp.int32),
                   jax.ShapeDtypeStruct((1, 1), jnp.float32)),
    )(xf, wp, bp)
    return out


def _convfields_body(xm_ref, xb_ref, xp_ref, cw_ref, tbl_ref, *, nsb):
    i = pl.program_id(1)
    xb = xb_ref[0]
    prev = jnp.concatenate([xm_ref[0, -1:], xb[:-1]], axis=0)
    nxt = jnp.concatenate([xb[1:], xp_ref[0, :1]], axis=0)
    rows = jax.lax.broadcasted_iota(jnp.int32, xb.shape, 0)
    prev = jnp.where((i == 0) & (rows == 0), 0.0, prev)
    nxt = jnp.where((i == nsb - 1) & (rows == xb.shape[0] - 1), 0.0, nxt)
    tbl_ref[0, 0] = xb
    for t in range(3):
        c = (prev * cw_ref[t, 0][None, :] + xb * cw_ref[t, 1][None, :]
             + nxt * cw_ref[t, 2][None, :])
        tbl_ref[t + 1, 0] = xb + c


def _convfields_call(x, conv_ws):
    B, S, D = x.shape
    BS = min(512, S)
    nsb = S // BS
    cw = jnp.stack([jnp.transpose(w[:, 0, :], (1, 0)) for w in conv_ws])
    grid = (B, nsb)
    out = pl.pallas_call(
        functools.partial(_convfields_body, nsb=nsb),
        grid=grid,
        in_specs=[
            pl.BlockSpec((1, BS, D), lambda b, i: (b, jnp.maximum(i - 1, 0), 0)),
            pl.BlockSpec((1, BS, D), lambda b, i: (b, i, 0)),
            pl.BlockSpec((1, BS, D), lambda b, i: (b, jnp.minimum(i + 1, nsb - 1), 0)),
            pl.BlockSpec((3, 3, D), lambda b, i: (0, 0, 0)),
        ],
        out_specs=pl.BlockSpec((4, 1, BS, D), lambda b, i: (0, b, i, 0)),
        out_shape=jax.ShapeDtypeStruct((4, B, S, D), jnp.float32),
    )(x, x, x, cw)
    return out.reshape(4, B * S, D)


def _mlp_body(x_ref, w1_ref, b1_ref, w2_ref, b2_ref, gw_ref, gb_ref, o_ref,
              *, nf, gated):
    f = pl.program_id(1)

    @pl.when(f == 0)
    def _():
        o_ref[...] = jnp.broadcast_to(b2_ref[...], o_ref.shape)

    x = x_ref[...]
    h = jax.lax.dot_general(x, w1_ref[...], (((1,), (0,)), ((), ())),
                            preferred_element_type=jnp.float32, precision=HI)
    h = _gelu(h + b1_ref[...])
    o_ref[...] += jax.lax.dot_general(h, w2_ref[...], (((1,), (0,)), ((), ())),
                                      preferred_element_type=jnp.float32,
                                      precision=HI)
    if gated:
        @pl.when(f == nf - 1)
        def _():
            g = jax.lax.dot_general(x, gw_ref[...], (((1,), (0,)), ((), ())),
                                    preferred_element_type=jnp.float32,
                                    precision=HI)
            g = jax.nn.sigmoid(g[:, :1] + gb_ref[0:1, 0:1])
            o_ref[...] *= g


def _mlp_call(xf, w1, b1, w2, b2, gate=None):
    M, D = xf.shape
    F = w1.shape[1]
    N = w2.shape[1]
    BM = min(256, M)
    BF = min(512, F)
    nf = F // BF
    gated = gate is not None
    if gated:
        gw, gb = gate
        gwp = jnp.zeros((D, 128), jnp.float32).at[:, :1].set(gw)
        gbp = jnp.full((1, 1), gb[0], jnp.float32)
    else:
        gwp = jnp.zeros((1, 128), jnp.float32)
        gbp = jnp.zeros((1, 1), jnp.float32)
    return pl.pallas_call(
        functools.partial(_mlp_body, nf=nf, gated=gated),
        grid=(M // BM, nf),
        in_specs=[
            pl.BlockSpec((BM, D), lambda m, f: (m, 0)),
            pl.BlockSpec((D, BF), lambda m, f: (0, f)),
            pl.BlockSpec((1, BF), lambda m, f: (0, f)),
            pl.BlockSpec((BF, N), lambda m, f: (f, 0)),
            pl.BlockSpec((1, N), lambda m, f: (0, 0)),
            pl.BlockSpec(gwp.shape, lambda m, f: (0, 0)),
            pl.BlockSpec((1, 1), lambda m, f: (0, 0)),
        ],
        out_specs=pl.BlockSpec((BM, N), lambda m, f: (m, 0)),
        out_shape=jax.ShapeDtypeStruct((M, N), jnp.float32),
    )(xf, w1, b1.reshape(1, F), w2, b2.reshape(1, N), gwp, gbp)


@functools.lru_cache(maxsize=2)
def _dft_consts(S):
    F = S // 2 + 1
    Fp = ((F + 127) // 128) * 128
    s = np.arange(S)
    f = np.arange(F)
    ang = 2.0 * np.pi * np.outer(f, s) / S
    CS = np.zeros((2 * Fp, S), np.float32)
    CS[:F] = np.cos(ang)
    CS[Fp:Fp + F] = -np.sin(ang)
    cr = np.full(F, 2.0); cr[0] = 1.0; cr[-1] = 1.0
    ci = np.full(F, 2.0); ci[0] = 0.0; ci[-1] = 0.0
    angT = ang.T
    CrCi = np.zeros((2, S, Fp), np.float32)
    CrCi[0, :, :F] = np.cos(angT) * cr / S
    CrCi[1, :, :F] = -np.sin(angT) * ci / S
    return CS, CrCi, Fp


def _matmul_body(a_ref, b_ref, o_ref, *, nk):
    k = pl.program_id(2)

    @pl.when(k == 0)
    def _():
        o_ref[...] = jnp.zeros_like(o_ref)

    o_ref[...] += jax.lax.dot_general(
        a_ref[...], b_ref[0], (((1,), (0,)), ((), ())),
        preferred_element_type=jnp.float32, precision=HI)


def _dft_call(x, CS):
    B, S, D = x.shape
    Fp2 = CS.shape[0]
    BM = min(256, Fp2)
    BK = min(1024, S)
    nk = S // BK
    out = pl.pallas_call(
        functools.partial(_matmul_body, nk=nk),
        grid=(B, Fp2 // BM, nk),
        in_specs=[
            pl.BlockSpec((BM, BK), lambda b, m, k: (m, k)),
            pl.BlockSpec((1, BK, D), lambda b, m, k: (b, k, 0)),
        ],
        out_specs=pl.BlockSpec((1, BM, D), lambda b, m, k: (b, m, 0)),
        out_shape=jax.ShapeDtypeStruct((B, Fp2, D), jnp.float32),
    )(CS, x)
    return out.reshape(B, 2, Fp2 // 2, D)


def _fmlp_body(ri_ref, w1_ref, b1_ref, w2_ref, b2_ref, o_ref, *, D):
    f = pl.program_id(2)

    @pl.when(f == 0)
    def _():
        o_ref[0, 0] = jnp.broadcast_to(b2_ref[:, :D], o_ref.shape[2:])
        o_ref[1, 0] = jnp.broadcast_to(b2_ref[:, D:], o_ref.shape[2:])

    re = ri_ref[0, 0]
    im = ri_ref[0, 1]
    h = jax.lax.dot_general(re, w1_ref[:D], (((1,), (0,)), ((), ())),
                            preferred_element_type=jnp.float32, precision=HI)
    h += jax.lax.dot_general(im, w1_ref[D:], (((1,), (0,)), ((), ())),
                             preferred_element_type=jnp.float32, precision=HI)
    h = _gelu(h + b1_ref[...])
    fo_re = jax.lax.dot_general(h, w2_ref[:, :D], (((1,), (0,)), ((), ())),
                                preferred_element_type=jnp.float32, precision=HI)
    fo_im = jax.lax.dot_general(h, w2_ref[:, D:], (((1,), (0,)), ((), ())),
                                preferred_element_type=jnp.float32, precision=HI)
    o_ref[0, 0] += fo_re
    o_ref[1, 0] += fo_im


def _fmlp_call(RI, w1, b1, w2, b2):
    B, _, Fp, D = RI.shape
    FF = w1.shape[1]
    BM = min(128, Fp)
    BF = min(512, FF)
    return pl.pallas_call(
        functools.partial(_fmlp_body, D=D),
        grid=(B, Fp // BM, FF // BF),
        in_specs=[
            pl.BlockSpec((1, 2, BM, D), lambda b, m, f: (b, 0, m, 0)),
            pl.BlockSpec((2 * D, BF), lambda b, m, f: (0, f)),
            pl.BlockSpec((1, BF), lambda b, m, f: (0, f)),
            pl.BlockSpec((BF, 2 * D), lambda b, m, f: (f, 0)),
            pl.BlockSpec((1, 2 * D), lambda b, m, f: (0, 0)),
        ],
        out_specs=pl.BlockSpec((2, 1, BM, D), lambda b, m, f: (0, b, m, 0)),
        out_shape=jax.ShapeDtypeStruct((2, B, Fp, D), jnp.float32),
    )(RI, w1, b1.reshape(1, FF), w2, b2.reshape(1, 2 * D))


def _irfft_body(c_ref, fo_ref, o_ref):
    p = pl.program_id(2)

    @pl.when(p == 0)
    def _():
        o_ref[...] = jnp.zeros_like(o_ref)

    o_ref[0] += jax.lax.dot_general(c_ref[0], fo_ref[0, 0],
                                    (((1,), (0,)), ((), ())),
                                    preferred_element_type=jnp.float32,
                                    precision=HI)


def _irfft_call(FO, CrCi):
    _, B, Fp, D = FO.shape
    S = CrCi.shape[1]
    BM = min(256, S)
    return pl.pallas_call(
        _irfft_body,
        grid=(B, S // BM, 2),
        in_specs=[
            pl.BlockSpec((1, BM, Fp), lambda b, s, p: (p, s, 0)),
            pl.BlockSpec((1, 1, Fp, D), lambda b, s, p: (p, b, 0, 0)),
        ],
        out_specs=pl.BlockSpec((1, BM, D), lambda b, s, p: (b, s, 0)),
        out_shape=jax.ShapeDtypeStruct((B, S, D), jnp.float32),
    )(CrCi, FO)


def _combine_body(base_ref, w_ref, j_ref, *rest):
    eo_refs = rest[:-1]
    o_ref = rest[-1]
    w0 = w_ref[:, 0:1]
    w1 = w_ref[:, 1:2]
    j0 = j_ref[:, 0:1]
    j1 = j_ref[:, 1:2]
    acc = base_ref[...]
    for j, eo in enumerate(eo_refs):
        coef = (jnp.where(j0 == j, w0, 0.0) + jnp.where(j1 == j, w1, 0.0))
        acc = acc + coef * eo[...]
    o_ref[...] = acc


def _combine_call(base, w01, jp, eos):
    M, D = base.shape
    BM = 256
    nspec = [pl.BlockSpec((BM, D), lambda m: (m, 0)),
             pl.BlockSpec((BM, 128), lambda m: (m, 0)),
             pl.BlockSpec((BM, 128), lambda m: (m, 0))]
    nspec += [pl.BlockSpec((BM, D), lambda m: (m, 0)) for _ in eos]
    return pl.pallas_call(
        _combine_body,
        grid=(M // BM,),
        in_specs=nspec,
        out_specs=pl.BlockSpec((BM, D), lambda m: (m, 0)),
        out_shape=jax.ShapeDtypeStruct((M, D), jnp.float32),
    )(base, w01, jp, *eos)


def kernel(x, params):
    B, S, D = x.shape
    E = params['router_b'].shape[0]
    xf = x.reshape(B * S, D)

    w01, jp, aux = _router_call(xf, params['router_w'], params['router_b'],
                                E, 2)

    conv_ws = [params['experts'][e]['conv_w'] for e in _SPARSE_EIDS
               if _EXPERT_TYPES[e] == 'conv']
    table = _convfields_call(x, conv_ws)

    base = _mlp_call(xf, params['shared_w1'], params['shared_b1'],
                     params['shared_w2'], params['shared_b2'],
                     gate=(params['gate_w'], params['gate_b']))

    CS_np, CrCi_np, Fp = _dft_consts(S)
    CS = jnp.asarray(CS_np)
    CrCi = jnp.asarray(CrCi_np)
    RI = _dft_call(x, CS)
    f_eos = []
    for e in _FOURIER_EIDS:
        p = params['experts'][e]
        FO = _fmlp_call(RI, p['w1'], p['b1'], p['w2'], p['b2'])
        f_eos.append(_irfft_call(FO, CrCi).reshape(B * S, D))

    s_eos = []
    conv_field = {0: 1, 3: 2, 6: 3}
    for e in _SPARSE_EIDS:
        p = params['experts'][e]
        inp = table[conv_field[e]] if _EXPERT_TYPES[e] == 'conv' else xf
        s_eos.append(_mlp_call(inp, p['w1'], p['b1'], p['w2'], p['b2']))

    out = _combine_call(base, w01, jp, s_eos + f_eos)
    return out.reshape(B, S, D), aux[0, 0]


# speedup vs baseline: 1.9094x; 1.9094x over previous
import functools
import math

import numpy as np
import jax
import jax.numpy as jnp
from jax.experimental import pallas as pl

HI = None

_EXPERT_TYPES = ['conv', 'fourier', 'mlp', 'conv', 'fourier', 'mlp', 'conv', 'fourier']
_SPARSE_EIDS = [0, 2, 3, 5, 6]
_FOURIER_EIDS = [1, 4, 7]
_JMAP = [0, 5, 1, 2, 6, 3, 4, 7]


def _gelu(h):
    return h * 0.5 * (1.0 + jax.lax.erf(h / np.float32(np.sqrt(2.0))))


def _router_body(xf, wp, bp, w01_ref, jp_ref, aux_ref, *, E, topk):
    x = xf[...]
    logits = jax.lax.dot_general(x, wp[...], (((1,), (0,)), ((), ())),
                                 preferred_element_type=jnp.float32) + bp[...]
    lanes = jax.lax.broadcasted_iota(jnp.int32, logits.shape, 1)
    neg = jnp.float32(-1e30)
    logits = jnp.where(lanes < E, logits, neg)
    m = jnp.max(logits, axis=-1, keepdims=True)
    ex = jnp.where(lanes < E, jnp.exp(logits - m), 0.0)
    probs = ex / jnp.sum(ex, axis=-1, keepdims=True)
    m1 = jnp.max(probs, axis=-1, keepdims=True)
    a1 = jnp.min(jnp.where(probs >= m1, lanes, E), axis=-1, keepdims=True)
    p2 = jnp.where(lanes == a1, neg, probs)
    m2 = jnp.max(p2, axis=-1, keepdims=True)
    a2 = jnp.min(jnp.where(p2 >= m2, lanes, E), axis=-1, keepdims=True)
    denom = m1 + m2
    w0 = m1 / denom
    w1 = m2 / denom
    col = lanes
    w01_ref[...] = jnp.where(col == 0, w0, 0.0) + jnp.where(col == 1, w1, 0.0)
    j1 = jnp.zeros_like(a1)
    j2 = jnp.zeros_like(a2)
    for e in range(E):
        j1 = j1 + jnp.where(a1 == e, _JMAP[e], 0)
        j2 = j2 + jnp.where(a2 == e, _JMAP[e], 0)
    jp_ref[...] = (jnp.where(col == 0, j1, 0) + jnp.where(col == 1, j2, 0)
                   ).astype(jnp.int32)
    ntok = x.shape[0]
    me = jnp.sum(probs, axis=0) / ntok
    cnt = jnp.sum((lanes == a1).astype(jnp.float32)
                  + (lanes == a2).astype(jnp.float32), axis=0)
    ce = cnt / (ntok * topk)
    aux_ref[...] = jnp.reshape(E * jnp.sum(me * ce), (1, 1))


def _router_call(xf, router_w, router_b, E, topk):
    M, D = xf.shape
    wp = jnp.zeros((D, 128), jnp.float32).at[:, :E].set(router_w)
    bp = jnp.zeros((1, 128), jnp.float32).at[0, :E].set(router_b)
    out = pl.pallas_call(
        functools.partial(_router_body, E=E, topk=topk),
        out_shape=(jax.ShapeDtypeStruct((M, 128), jnp.float32),
                   jax.ShapeDtypeStruct((M, 128), jnp.int32),
                   jax.ShapeDtypeStruct((1, 1), jnp.float32)),
    )(xf, wp, bp)
    return out


def _convfields_body(xm_ref, xb_ref, xp_ref, cw_ref, tbl_ref, *, nsb):
    i = pl.program_id(1)
    xb = xb_ref[0]
    prev = jnp.concatenate([xm_ref[0, -1:], xb[:-1]], axis=0)
    nxt = jnp.concatenate([xb[1:], xp_ref[0, :1]], axis=0)
    rows = jax.lax.broadcasted_iota(jnp.int32, xb.shape, 0)
    prev = jnp.where((i == 0) & (rows == 0), 0.0, prev)
    nxt = jnp.where((i == nsb - 1) & (rows == xb.shape[0] - 1), 0.0, nxt)
    tbl_ref[0, 0] = xb
    for t in range(3):
        c = (prev * cw_ref[t, 0][None, :] + xb * cw_ref[t, 1][None, :]
             + nxt * cw_ref[t, 2][None, :])
        tbl_ref[t + 1, 0] = xb + c


def _convfields_call(x, conv_ws):
    B, S, D = x.shape
    BS = min(512, S)
    nsb = S // BS
    cw = jnp.stack([jnp.transpose(w[:, 0, :], (1, 0)) for w in conv_ws])
    grid = (B, nsb)
    out = pl.pallas_call(
        functools.partial(_convfields_body, nsb=nsb),
        grid=grid,
        in_specs=[
            pl.BlockSpec((1, BS, D), lambda b, i: (b, jnp.maximum(i - 1, 0), 0)),
            pl.BlockSpec((1, BS, D), lambda b, i: (b, i, 0)),
            pl.BlockSpec((1, BS, D), lambda b, i: (b, jnp.minimum(i + 1, nsb - 1), 0)),
            pl.BlockSpec((3, 3, D), lambda b, i: (0, 0, 0)),
        ],
        out_specs=pl.BlockSpec((4, 1, BS, D), lambda b, i: (0, b, i, 0)),
        out_shape=jax.ShapeDtypeStruct((4, B, S, D), jnp.float32),
    )(x, x, x, cw)
    return out.reshape(4, B * S, D)


def _mlp_body(x_ref, w1_ref, b1_ref, w2_ref, b2_ref, gw_ref, gb_ref, o_ref,
              *, nf, gated):
    f = pl.program_id(1)

    @pl.when(f == 0)
    def _():
        o_ref[...] = jnp.broadcast_to(b2_ref[...], o_ref.shape)

    x = x_ref[...]
    h = jax.lax.dot_general(x, w1_ref[...], (((1,), (0,)), ((), ())),
                            preferred_element_type=jnp.float32, precision=HI)
    h = _gelu(h + b1_ref[...])
    o_ref[...] += jax.lax.dot_general(h, w2_ref[...], (((1,), (0,)), ((), ())),
                                      preferred_element_type=jnp.float32,
                                      precision=HI)
    if gated:
        @pl.when(f == nf - 1)
        def _():
            g = jax.lax.dot_general(x, gw_ref[...], (((1,), (0,)), ((), ())),
                                    preferred_element_type=jnp.float32,
                                    precision=HI)
            g = jax.nn.sigmoid(g[:, :1] + gb_ref[0:1, 0:1])
            o_ref[...] *= g


def _mlp_call(xf, w1, b1, w2, b2, gate=None):
    M, D = xf.shape
    F = w1.shape[1]
    N = w2.shape[1]
    BM = min(256, M)
    BF = min(512, F)
    nf = F // BF
    gated = gate is not None
    if gated:
        gw, gb = gate
        gwp = jnp.zeros((D, 128), jnp.float32).at[:, :1].set(gw)
        gbp = jnp.full((1, 1), gb[0], jnp.float32)
    else:
        gwp = jnp.zeros((1, 128), jnp.float32)
        gbp = jnp.zeros((1, 1), jnp.float32)
    return pl.pallas_call(
        functools.partial(_mlp_body, nf=nf, gated=gated),
        grid=(M // BM, nf),
        in_specs=[
            pl.BlockSpec((BM, D), lambda m, f: (m, 0)),
            pl.BlockSpec((D, BF), lambda m, f: (0, f)),
            pl.BlockSpec((1, BF), lambda m, f: (0, f)),
            pl.BlockSpec((BF, N), lambda m, f: (f, 0)),
            pl.BlockSpec((1, N), lambda m, f: (0, 0)),
            pl.BlockSpec(gwp.shape, lambda m, f: (0, 0)),
            pl.BlockSpec((1, 1), lambda m, f: (0, 0)),
        ],
        out_specs=pl.BlockSpec((BM, N), lambda m, f: (m, 0)),
        out_shape=jax.ShapeDtypeStruct((M, N), jnp.float32),
    )(xf, w1, b1.reshape(1, F), w2, b2.reshape(1, N), gwp, gbp)


@functools.lru_cache(maxsize=2)
def _dft_consts(S):
    F = S // 2 + 1
    Fp = ((F + 127) // 128) * 128
    s = np.arange(S)
    f = np.arange(F)
    ang = 2.0 * np.pi * np.outer(f, s) / S
    CS = np.zeros((2 * Fp, S), np.float32)
    CS[:F] = np.cos(ang)
    CS[Fp:Fp + F] = -np.sin(ang)
    cr = np.full(F, 2.0); cr[0] = 1.0; cr[-1] = 1.0
    ci = np.full(F, 2.0); ci[0] = 0.0; ci[-1] = 0.0
    angT = ang.T
    CrCi = np.zeros((2, S, Fp), np.float32)
    CrCi[0, :, :F] = np.cos(angT) * cr / S
    CrCi[1, :, :F] = -np.sin(angT) * ci / S
    return CS, CrCi, Fp


def _matmul_body(a_ref, b_ref, o_ref, *, nk):
    k = pl.program_id(2)

    @pl.when(k == 0)
    def _():
        o_ref[...] = jnp.zeros_like(o_ref)

    o_ref[...] += jax.lax.dot_general(
        a_ref[...], b_ref[0], (((1,), (0,)), ((), ())),
        preferred_element_type=jnp.float32, precision=HI)


def _dft_call(x, CS):
    B, S, D = x.shape
    Fp2 = CS.shape[0]
    BM = min(256, Fp2)
    BK = min(1024, S)
    nk = S // BK
    out = pl.pallas_call(
        functools.partial(_matmul_body, nk=nk),
        grid=(B, Fp2 // BM, nk),
        in_specs=[
            pl.BlockSpec((BM, BK), lambda b, m, k: (m, k)),
            pl.BlockSpec((1, BK, D), lambda b, m, k: (b, k, 0)),
        ],
        out_specs=pl.BlockSpec((1, BM, D), lambda b, m, k: (b, m, 0)),
        out_shape=jax.ShapeDtypeStruct((B, Fp2, D), jnp.float32),
    )(CS, x)
    return out.reshape(B, 2, Fp2 // 2, D)


def _fmlp_body(ri_ref, w1_ref, b1_ref, w2_ref, b2_ref, o_ref, *, D):
    f = pl.program_id(2)

    @pl.when(f == 0)
    def _():
        o_ref[0, 0] = jnp.broadcast_to(b2_ref[:, :D], o_ref.shape[2:])
        o_ref[1, 0] = jnp.broadcast_to(b2_ref[:, D:], o_ref.shape[2:])

    re = ri_ref[0, 0]
    im = ri_ref[0, 1]
    h = jax.lax.dot_general(re, w1_ref[:D], (((1,), (0,)), ((), ())),
                            preferred_element_type=jnp.float32, precision=HI)
    h += jax.lax.dot_general(im, w1_ref[D:], (((1,), (0,)), ((), ())),
                             preferred_element_type=jnp.float32, precision=HI)
    h = _gelu(h + b1_ref[...])
    fo_re = jax.lax.dot_general(h, w2_ref[:, :D], (((1,), (0,)), ((), ())),
                                preferred_element_type=jnp.float32, precision=HI)
    fo_im = jax.lax.dot_general(h, w2_ref[:, D:], (((1,), (0,)), ((), ())),
                                preferred_element_type=jnp.float32, precision=HI)
    o_ref[0, 0] += fo_re
    o_ref[1, 0] += fo_im


def _fmlp_call(RI, w1, b1, w2, b2):
    B, _, Fp, D = RI.shape
    FF = w1.shape[1]
    BM = min(128, Fp)
    BF = min(512, FF)
    return pl.pallas_call(
        functools.partial(_fmlp_body, D=D),
        grid=(B, Fp // BM, FF // BF),
        in_specs=[
            pl.BlockSpec((1, 2, BM, D), lambda b, m, f: (b, 0, m, 0)),
            pl.BlockSpec((2 * D, BF), lambda b, m, f: (0, f)),
            pl.BlockSpec((1, BF), lambda b, m, f: (0, f)),
            pl.BlockSpec((BF, 2 * D), lambda b, m, f: (f, 0)),
            pl.BlockSpec((1, 2 * D), lambda b, m, f: (0, 0)),
        ],
        out_specs=pl.BlockSpec((2, 1, BM, D), lambda b, m, f: (0, b, m, 0)),
        out_shape=jax.ShapeDtypeStruct((2, B, Fp, D), jnp.float32),
    )(RI, w1, b1.reshape(1, FF), w2, b2.reshape(1, 2 * D))


def _irfft_body(c_ref, fo_ref, o_ref):
    p = pl.program_id(2)

    @pl.when(p == 0)
    def _():
        o_ref[...] = jnp.zeros_like(o_ref)

    o_ref[0] += jax.lax.dot_general(c_ref[0], fo_ref[0, 0],
                                    (((1,), (0,)), ((), ())),
                                    preferred_element_type=jnp.float32,
                                    precision=HI)


def _irfft_call(FO, CrCi):
    _, B, Fp, D = FO.shape
    S = CrCi.shape[1]
    BM = min(256, S)
    return pl.pallas_call(
        _irfft_body,
        grid=(B, S // BM, 2),
        in_specs=[
            pl.BlockSpec((1, BM, Fp), lambda b, s, p: (p, s, 0)),
            pl.BlockSpec((1, 1, Fp, D), lambda b, s, p: (p, b, 0, 0)),
        ],
        out_specs=pl.BlockSpec((1, BM, D), lambda b, s, p: (b, s, 0)),
        out_shape=jax.ShapeDtypeStruct((B, S, D), jnp.float32),
    )(CrCi, FO)


def _combine_body(base_ref, w_ref, j_ref, *rest):
    eo_refs = rest[:-1]
    o_ref = rest[-1]
    w0 = w_ref[:, 0:1]
    w1 = w_ref[:, 1:2]
    j0 = j_ref[:, 0:1]
    j1 = j_ref[:, 1:2]
    acc = base_ref[...]
    for j, eo in enumerate(eo_refs):
        coef = (jnp.where(j0 == j, w0, 0.0) + jnp.where(j1 == j, w1, 0.0))
        acc = acc + coef * eo[...]
    o_ref[...] = acc


def _combine_call(base, w01, jp, eos):
    M, D = base.shape
    BM = 256
    nspec = [pl.BlockSpec((BM, D), lambda m: (m, 0)),
             pl.BlockSpec((BM, 128), lambda m: (m, 0)),
             pl.BlockSpec((BM, 128), lambda m: (m, 0))]
    nspec += [pl.BlockSpec((BM, D), lambda m: (m, 0)) for _ in eos]
    return pl.pallas_call(
        _combine_body,
        grid=(M // BM,),
        in_specs=nspec,
        out_specs=pl.BlockSpec((BM, D), lambda m: (m, 0)),
        out_shape=jax.ShapeDtypeStruct((M, D), jnp.float32),
    )(base, w01, jp, *eos)


def kernel(x, params):
    B, S, D = x.shape
    E = params['router_b'].shape[0]
    xf = x.reshape(B * S, D)

    w01, jp, aux = _router_call(xf, params['router_w'], params['router_b'],
                                E, 2)

    conv_ws = [params['experts'][e]['conv_w'] for e in _SPARSE_EIDS
               if _EXPERT_TYPES[e] == 'conv']
    table = _convfields_call(x, conv_ws)

    base = _mlp_call(xf, params['shared_w1'], params['shared_b1'],
                     params['shared_w2'], params['shared_b2'],
                     gate=(params['gate_w'], params['gate_b']))

    CS_np, CrCi_np, Fp = _dft_consts(S)
    CS = jnp.asarray(CS_np)
    CrCi = jnp.asarray(CrCi_np)
    RI = _dft_call(x, CS)
    f_eos = []
    for e in _FOURIER_EIDS:
        p = params['experts'][e]
        FO = _fmlp_call(RI, p['w1'], p['b1'], p['w2'], p['b2'])
        f_eos.append(_irfft_call(FO, CrCi).reshape(B * S, D))

    s_eos = []
    conv_field = {0: 1, 3: 2, 6: 3}
    for e in _SPARSE_EIDS:
        p = params['experts'][e]
        inp = table[conv_field[e]] if _EXPERT_TYPES[e] == 'conv' else xf
        s_eos.append(_mlp_call(inp, p['w1'], p['b1'], p['w2'], p['b2']))

    out = _combine_call(base, w01, jp, s_eos + f_eos)
    return out.reshape(B, S, D), aux[0, 0]
